# Initial kernel scaffold; baseline (speedup 1.0000x reference)
import functools

import jax
import jax.numpy as jnp
from jax import lax
from jax.experimental import pallas as pl
from jax.experimental.pallas import tpu as pltpu
from jax.experimental.pallas import tpu_sc as plsc

N = 10000
D = 128
E = 320000
OUT = 3

NC = 2
NS = 16
NW = NC * NS
CHUNK = 128
CPW = -(-E // (NW * CHUNK))
E_PAD = NW * CPW * CHUNK
N_TRASH = 16
N_PAD = N + N_TRASH
RPT = N_PAD // NS

MBLK = 2000


def _make_sc_agg():
    mesh = plsc.VectorSubcoreMesh(core_axis_name="c", subcore_axis_name="s")

    @functools.partial(
        pl.kernel,
        out_type=jax.ShapeDtypeStruct((NC, N_PAD, D), jnp.float32),
        mesh=mesh,
        scratch_types=[
            pltpu.VMEM((CPW, CHUNK), jnp.int32),
            pltpu.VMEM((CPW, CHUNK), jnp.int32),
            pltpu.VMEM((2, CHUNK, D), jnp.float32),
            pltpu.VMEM_SHARED((N_PAD, D), jnp.float32),
            pltpu.SemaphoreType.DMA,
            pltpu.SemaphoreType.DMA,
        ],
    )
    def sc_agg(src_hbm, dst_hbm, zeros_hbm, h_hbm, out_hbm,
               src_v, dst_v, rows_v, agg_sh, sem_g, sem_s):
        c = lax.axis_index("c")
        s = lax.axis_index("s")
        wid = c * NS + s
        pltpu.sync_copy(zeros_hbm, agg_sh.at[pl.ds(s * RPT, RPT)])
        pltpu.sync_copy(src_hbm.at[wid], src_v)
        pltpu.sync_copy(dst_hbm.at[wid], dst_v)
        plsc.subcore_barrier()

        gather = lambda j, b: pltpu.async_copy(
            h_hbm.at[src_v.at[j]], rows_v.at[b], sem_g)
        scat = lambda j, b: pltpu.async_copy(
            rows_v.at[b], agg_sh.at[dst_v.at[j]], sem_s, add=True)

        gather(0, 0)

        @pl.loop(0, CPW - 1)
        def _(j):
            b = lax.rem(j, 2)
            pltpu.make_async_copy(h_hbm.at[src_v.at[j]], rows_v.at[b], sem_g).wait()
            gather(j + 1, 1 - b)
            scat(j, b).wait()

        last = CPW - 1
        b = (CPW - 1) % 2
        pltpu.make_async_copy(h_hbm.at[src_v.at[last]], rows_v.at[b], sem_g).wait()
        scat(last, b).wait()

        plsc.subcore_barrier()
        pltpu.sync_copy(agg_sh.at[pl.ds(s * RPT, RPT)],
                        out_hbm.at[c, pl.ds(s * RPT, RPT)])

    return sc_agg


_sc_agg = _make_sc_agg()


def _tc_body(relu, avg, h_ref, p_ref, ws_ref, wn_ref, b_ref, *rest):
    if avg:
        temp_ref, o_ref = rest
    else:
        (o_ref,) = rest
    agg = p_ref[0] + p_ref[1]
    y = (jnp.dot(h_ref[...], ws_ref[...], preferred_element_type=jnp.float32)
         + jnp.dot(agg, wn_ref[...], preferred_element_type=jnp.float32)
         + b_ref[...])
    if relu:
        y = jnp.maximum(y, 0.0)
    if avg:
        y = (temp_ref[...] + y) * 0.5
    o_ref[...] = y


def _make_tc(relu, avg):
    nblk = N // MBLK
    in_specs = [
        pl.BlockSpec((MBLK, D), lambda i: (i, 0)),
        pl.BlockSpec((NC, MBLK, D), lambda i: (0, i, 0)),
        pl.BlockSpec((D, D), lambda i: (0, 0)),
        pl.BlockSpec((D, D), lambda i: (0, 0)),
        pl.BlockSpec((1, D), lambda i: (0, 0)),
    ]
    if avg:
        in_specs.append(pl.BlockSpec((MBLK, D), lambda i: (i, 0)))
    return pl.pallas_call(
        functools.partial(_tc_body, relu, avg),
        grid=(nblk,),
        in_specs=in_specs,
        out_specs=pl.BlockSpec((MBLK, D), lambda i: (i, 0)),
        out_shape=jax.ShapeDtypeStruct((N, D), jnp.float32),
    )


_tc_relu = _make_tc(True, False)
_tc_relu_avg = _make_tc(True, True)
_tc_plain = _make_tc(False, False)


def kernel(edges, shape_features, Ws, Wn, bs, Wout_s, Wout_n, b_out):
    src = edges[0]
    dst = edges[1]
    pad = E_PAD - E
    pad_ids = lax.iota(jnp.int32, pad)
    src3 = jnp.concatenate([src, pad_ids % N]).reshape(NW, CPW, CHUNK)
    dst3 = jnp.concatenate([dst, N + (pad_ids % N_TRASH)]).reshape(NW, CPW, CHUNK)
    zeros = jnp.zeros((RPT, D), jnp.float32)

    def gconv(h, W_s, W_n, b, temp=None, relu=True):
        p = _sc_agg(src3, dst3, zeros, h)
        b2 = b.reshape(1, D)
        if temp is not None:
            return _tc_relu_avg(h, p, W_s, W_n, b2, temp)
        if relu:
            return _tc_relu(h, p, W_s, W_n, b2)
        return _tc_plain(h, p, W_s, W_n, b2)

    h = gconv(shape_features, Ws[0], Wn[0], bs[0])
    for i in range(1, 12, 2):
        temp = h
        h = gconv(h, Ws[i], Wn[i], bs[i])
        h = gconv(h, Ws[i + 1], Wn[i + 1], bs[i + 1], temp=temp)

    Wo_s = jnp.zeros((D, D), jnp.float32).at[:, :OUT].set(Wout_s)
    Wo_n = jnp.zeros((D, D), jnp.float32).at[:, :OUT].set(Wout_n)
    bo = jnp.zeros((D,), jnp.float32).at[:OUT].set(b_out)
    coords = gconv(h, Wo_s, Wo_n, bo, relu=False)[:, :OUT]
    return (h, coords)


# speedup vs baseline: 10.6223x; 10.6223x over previous
import functools

import jax
import jax.numpy as jnp
from jax import lax
from jax.experimental import pallas as pl
from jax.experimental.pallas import tpu as pltpu
from jax.experimental.pallas import tpu_sc as plsc

N = 10000
D = 128
E = 320000
OUT = 3

NC = 2
NS = 16
NW = NC * NS
CHUNK = 128
NBUF = 3
CPW = 81
NGRP = CPW // NBUF
E_PAD = NW * CPW * CHUNK
N_TRASH = 112
N_PAD = N + N_TRASH
RPT = N_PAD // NS

MBLK = 2000


def _make_sc_agg():
    mesh = plsc.VectorSubcoreMesh(core_axis_name="c", subcore_axis_name="s")

    @functools.partial(
        pl.kernel,
        out_type=jax.ShapeDtypeStruct((NC, N_PAD, D), jnp.float32),
        mesh=mesh,
        scratch_types=[
            pltpu.VMEM((NBUF, 2, CHUNK), jnp.int32),
            pltpu.VMEM((NBUF, CHUNK, D), jnp.float32),
            pltpu.VMEM_SHARED((N_PAD, D), jnp.float32),
            pltpu.SemaphoreType.DMA((NBUF,)),
            pltpu.SemaphoreType.DMA((NBUF,)),
            pltpu.SemaphoreType.DMA((NBUF,)),
        ],
    )
    def sc_agg(eidx_hbm, zeros_hbm, h_hbm, out_hbm,
               idx_v, rows_v, agg_sh, sem_i, sem_g, sem_s):
        c = lax.axis_index("c")
        s = lax.axis_index("s")
        wid = c * NS + s

        def idx_load(j, b):
            return pltpu.make_async_copy(
                eidx_hbm.at[wid, j], idx_v.at[b], sem_i.at[b])

        def gath(j, b):
            return pltpu.make_async_copy(
                h_hbm.at[idx_v.at[b, 0]], rows_v.at[b], sem_g.at[b])

        def scat(b):
            return pltpu.make_async_copy(
                rows_v.at[b], agg_sh.at[idx_v.at[b, 1]], sem_s.at[b])

        for b in range(NBUF):
            idx_load(b, b).start()
        pltpu.sync_copy(zeros_hbm, agg_sh.at[pl.ds(s * RPT, RPT)])
        plsc.subcore_barrier()
        for b in range(NBUF):
            idx_load(b, b).wait()
            gath(b, b).start()

        @pl.loop(0, NGRP - 1)
        def _(g):
            for b in range(NBUF):
                j = g * NBUF + b
                gath(j, b).wait()
                pltpu.async_copy(rows_v.at[b], agg_sh.at[idx_v.at[b, 1]],
                                 sem_s.at[b], add=True)
                scat(b).wait()
                idx_load(j + NBUF, b).start()
                idx_load(j + NBUF, b).wait()
                gath(j + NBUF, b).start()

        for b in range(NBUF):
            j = (NGRP - 1) * NBUF + b
            gath(j, b).wait()
            pltpu.async_copy(rows_v.at[b], agg_sh.at[idx_v.at[b, 1]],
                             sem_s.at[b], add=True)
            scat(b).wait()

        plsc.subcore_barrier()
        pltpu.sync_copy(agg_sh.at[pl.ds(s * RPT, RPT)],
                        out_hbm.at[c, pl.ds(s * RPT, RPT)])

    return sc_agg


_sc_agg = _make_sc_agg()


def _tc_body(relu, avg, h_ref, p_ref, ws_ref, wn_ref, b_ref, *rest):
    if avg:
        temp_ref, o_ref = rest
    else:
        (o_ref,) = rest
    agg = p_ref[0] + p_ref[1]
    y = (jnp.dot(h_ref[...], ws_ref[...], preferred_element_type=jnp.float32)
         + jnp.dot(agg, wn_ref[...], preferred_element_type=jnp.float32)
         + b_ref[...])
    if relu:
        y = jnp.maximum(y, 0.0)
    if avg:
        y = (temp_ref[...] + y) * 0.5
    o_ref[...] = y


def _make_tc(relu, avg):
    nblk = N // MBLK
    in_specs = [
        pl.BlockSpec((MBLK, D), lambda i: (i, 0)),
        pl.BlockSpec((NC, MBLK, D), lambda i: (0, i, 0)),
        pl.BlockSpec((D, D), lambda i: (0, 0)),
        pl.BlockSpec((D, D), lambda i: (0, 0)),
        pl.BlockSpec((1, D), lambda i: (0, 0)),
    ]
    if avg:
        in_specs.append(pl.BlockSpec((MBLK, D), lambda i: (i, 0)))
    return pl.pallas_call(
        functools.partial(_tc_body, relu, avg),
        grid=(nblk,),
        in_specs=in_specs,
        out_specs=pl.BlockSpec((MBLK, D), lambda i: (i, 0)),
        out_shape=jax.ShapeDtypeStruct((N, D), jnp.float32),
    )


_tc_relu = _make_tc(True, False)
_tc_relu_avg = _make_tc(True, True)
_tc_plain = _make_tc(False, False)


def kernel(edges, shape_features, Ws, Wn, bs, Wout_s, Wout_n, b_out):
    src = edges[0]
    dst = edges[1]
    pad = E_PAD - E
    pad_ids = lax.iota(jnp.int32, pad)
    src3 = jnp.concatenate([src, pad_ids % N]).reshape(NW, CPW, CHUNK)
    dst3 = jnp.concatenate([dst, N + (pad_ids % N_TRASH)]).reshape(NW, CPW, CHUNK)
    eidx = jnp.stack([src3, dst3], axis=2)
    zeros = jnp.zeros((RPT, D), jnp.float32)

    def gconv(h, W_s, W_n, b, temp=None, relu=True):
        p = _sc_agg(eidx, zeros, h)
        b2 = b.reshape(1, D)
        if temp is not None:
            return _tc_relu_avg(h, p, W_s, W_n, b2, temp)
        if relu:
            return _tc_relu(h, p, W_s, W_n, b2)
        return _tc_plain(h, p, W_s, W_n, b2)

    h = gconv(shape_features, Ws[0], Wn[0], bs[0])
    for i in range(1, 12, 2):
        temp = h
        h = gconv(h, Ws[i], Wn[i], bs[i])
        h = gconv(h, Ws[i + 1], Wn[i + 1], bs[i + 1], temp=temp)

    Wo_s = jnp.zeros((D, D), jnp.float32).at[:, :OUT].set(Wout_s)
    Wo_n = jnp.zeros((D, D), jnp.float32).at[:, :OUT].set(Wout_n)
    bo = jnp.zeros((D,), jnp.float32).at[:OUT].set(b_out)
    coords = gconv(h, Wo_s, Wo_n, bo, relu=False)[:, :OUT]
    return (h, coords)
